# baseline (speedup 1.0000x reference)
import functools

import jax
import jax.numpy as jnp
from jax import lax
from jax.experimental import pallas as pl
from jax.experimental.pallas import tpu as pltpu
from jax.experimental.pallas import tpu_sc as plsc

_ALPHA = 0.7
_KFRAC = 0.2
_L = 16
_NB = 64


def _positive_k(k, n):
    if k <= 0:
        return 0
    elif k < 1:
        return int(round(float(n) * float(k)))
    elif k > n:
        return int(n)
    else:
        return int(k)


def _keyu(v):
    i = lax.bitcast_convert_type(v, jnp.int32)
    return jnp.where(i < 0, ~i, i ^ jnp.int32(-2**31))


def _val_of_u(u):
    i = jnp.where(u < 0, u ^ jnp.int32(-2**31), ~u)
    return lax.bitcast_convert_type(i, jnp.float32)


def _digit(u, shift):
    s = jnp.full((_L,), shift, jnp.int32)
    return lax.shift_right_logical(u, s) & jnp.int32(_NB - 1)


def _sc_pool_kernel_body(n, kmax, kmin, B, C,
                         x_hbm, out_hbm,
                         data_v, listt_v, listb_v, hcnt_v, hsum_v, outb_v,
                         sem):
    lanes = lax.iota(jnp.int32, _L)
    zi = jnp.zeros((_L,), jnp.int32)
    zf = jnp.zeros((_L,), jnp.float32)
    ones_i = jnp.ones((_L,), jnp.int32)

    ncores = 2
    nsub = 16
    wid = lax.axis_index("c") * nsub + lax.axis_index("s")
    ngroups = B * (C // _L)
    gpw = ngroups // (ncores * nsub)

    def zero_h(d, _):
        hcnt_v[d] = zi
        hsum_v[d] = zf
        return 0
    lax.fori_loop(0, _NB, zero_h, 0)

    def scan_top(m_vec, krem):
        thresh = m_vec - krem + 1

        def body(d, carry):
            cum_c, cum_s, crossed, D, cntle, sumle, bcnt = carry
            c = hcnt_v[d]
            s = hsum_v[d]
            hcnt_v[d] = zi
            hsum_v[d] = zf
            cum_c = cum_c + c
            cum_s = cum_s + s
            newly = jnp.logical_and(jnp.logical_not(crossed), cum_c >= thresh)
            dv = zi + d
            D = jnp.where(newly, dv, D)
            cntle = jnp.where(newly, cum_c, cntle)
            sumle = jnp.where(newly, cum_s, sumle)
            bcnt = jnp.where(newly, c, bcnt)
            crossed = jnp.logical_or(crossed, newly)
            return cum_c, cum_s, crossed, D, cntle, sumle, bcnt

        init = (zi, zf, zi > 0, zi, zi, zf, zi)
        cum_c, cum_s, _, D, cntle, sumle, bcnt = lax.fori_loop(
            0, _NB, body, init)
        add_cnt = m_vec - cntle
        add_sum = cum_s - sumle
        return D, add_cnt, add_sum, bcnt

    def scan_bot(m_vec, krem):
        thresh = krem

        def body(d, carry):
            cum_c, cum_s, crossed, D, cntlt, sumlt, bcnt = carry
            c = hcnt_v[d]
            s = hsum_v[d]
            hcnt_v[d] = zi
            hsum_v[d] = zf
            cum_c = cum_c + c
            cum_s = cum_s + s
            newly = jnp.logical_and(jnp.logical_not(crossed), cum_c >= thresh)
            dv = zi + d
            D = jnp.where(newly, dv, D)
            cntlt = jnp.where(newly, cum_c - c, cntlt)
            sumlt = jnp.where(newly, cum_s - s, sumlt)
            bcnt = jnp.where(newly, c, bcnt)
            crossed = jnp.logical_or(crossed, newly)
            return cum_c, cum_s, crossed, D, cntlt, sumlt, bcnt

        init = (zi, zf, zi > 0, zi, zi, zf, zi)
        _, _, _, D, cntlt, sumlt, bcnt = lax.fori_loop(0, _NB, body, init)
        return D, cntlt, sumlt, bcnt

    def gather_row(list_v, j, m_vec):
        jv = zi + j
        valid = jv < m_vec
        idx = jnp.where(valid, list_v[j], 0)
        v = plsc.load_gather(data_v, [idx, lanes])
        return idx, v, valid

    def hist_list(list_v, m_vec, shift):
        jmax = jnp.max(m_vec)

        def body(j, _):
            _, v, valid = gather_row(list_v, j, m_vec)
            u = _keyu(v)
            d = jnp.where(valid, _digit(u, shift), 0)
            plsc.addupdate_scatter(hcnt_v, [d, lanes],
                                   jnp.where(valid, 1, 0).astype(jnp.int32))
            plsc.addupdate_scatter(hsum_v, [d, lanes],
                                   jnp.where(valid, v, 0.0))
            return 0

        lax.fori_loop(0, jmax, body, 0)

    def compact_list(list_v, m_vec, D, shift):
        jmax = jnp.max(m_vec)

        def body(j, cur):
            idx, v, valid = gather_row(list_v, j, m_vec)
            u = _keyu(v)
            d = _digit(u, shift)
            mask = jnp.logical_and(valid, d == D)
            plsc.store_scatter(list_v, [cur, lanes], idx, mask=mask)
            return cur + jnp.where(mask, 1, 0)

        lax.fori_loop(0, jmax, body, zi)

    def descent(list_v, m_vec, prefix, krank):
        jmax = jnp.max(m_vec)

        def bitbody(bi, t):
            bit = lax.shift_right_arithmetic(jnp.int32(1 << 13), bi)
            cand = t | bit

            def cntbody(j, c):
                _, v, valid = gather_row(list_v, j, m_vec)
                u = _keyu(v)
                hit = jnp.logical_and(valid, u >= cand)
                return c + jnp.where(hit, 1, 0)

            c = lax.fori_loop(0, jmax, cntbody, zi)
            return jnp.where(c >= krank, cand, t)

        t = lax.fori_loop(0, 14, bitbody, prefix)

        def gtbody(j, carry):
            cg, sg = carry
            _, v, valid = gather_row(list_v, j, m_vec)
            u = _keyu(v)
            g = jnp.logical_and(valid, u > t)
            return cg + jnp.where(g, 1, 0), sg + jnp.where(g, v, 0.0)

        cnt_gt, sum_gt = lax.fori_loop(0, jmax, gtbody, (zi, zf))
        return t, cnt_gt, sum_gt

    def ltstats(list_v, m_vec, t):
        jmax = jnp.max(m_vec)

        def body(j, carry):
            cl, sl = carry
            _, v, valid = gather_row(list_v, j, m_vec)
            u = _keyu(v)
            g = jnp.logical_and(valid, u < t)
            return cl + jnp.where(g, 1, 0), sl + jnp.where(g, v, 0.0)

        return lax.fori_loop(0, jmax, body, (zi, zf))

    cpg = C // _L

    def group_body(g, _):
        gid = wid * gpw + g
        b = gid // cpg
        c0 = (gid % cpg) * _L

        pltpu.sync_copy(x_hbm.at[b, :, pl.ds(c0, _L)], data_v)

        def histA(i, _):
            v = data_v[i]
            u = _keyu(v)
            d = _digit(u, 26)
            plsc.addupdate_scatter(hcnt_v, [d, lanes], ones_i)
            plsc.addupdate_scatter(hsum_v, [d, lanes], v)
            return 0

        lax.fori_loop(0, n, histA, 0)

        m0 = zi + n
        k_t = zi + kmax
        k_b = zi + kmin
        D_t, add_c, add_s, bcnt_t = scan_top(m0, k_t)
        sel_c_t = add_c
        sel_s_t = add_s
        krem_t = k_t - add_c
        m_t = bcnt_t
        D_b, cntlt, sumlt, bcnt_b = scan_bot(m0, k_b)
        sel_c_b = cntlt
        sel_s_b = sumlt
        krem_b = k_b - cntlt
        m_b = bcnt_b

        def compactC(i, carry):
            cur_t, cur_b = carry
            v = data_v[i]
            u = _keyu(v)
            d = _digit(u, 26)
            iv = zi + i
            mt = d == D_t
            plsc.store_scatter(listt_v, [cur_t, lanes], iv, mask=mt)
            mb = d == D_b
            plsc.store_scatter(listb_v, [cur_b, lanes], iv, mask=mb)
            return cur_t + jnp.where(mt, 1, 0), cur_b + jnp.where(mb, 1, 0)

        lax.fori_loop(0, n, compactC, (zi, zi))

        Dt1, Db1 = D_t, D_b
        Dts = [Dt1]
        Dbs = [Db1]
        for shift in (20, 14):
            hist_list(listt_v, m_t, shift)
            D, add_c, add_s, bcnt = scan_top(m_t, krem_t)
            sel_c_t = sel_c_t + add_c
            sel_s_t = sel_s_t + add_s
            krem_t = krem_t - add_c
            compact_list(listt_v, m_t, D, shift)
            m_t = bcnt
            Dts.append(D)

            hist_list(listb_v, m_b, shift)
            D, cntlt, sumlt, bcnt = scan_bot(m_b, krem_b)
            sel_c_b = sel_c_b + cntlt
            sel_s_b = sel_s_b + sumlt
            krem_b = krem_b - cntlt
            compact_list(listb_v, m_b, D, shift)
            m_b = bcnt
            Dbs.append(D)

        six = jnp.int32(6)
        pre_t = ((Dts[0] << six | Dts[1]) << six | Dts[2]) << jnp.int32(14)
        pre_b = ((Dbs[0] << six | Dbs[1]) << six | Dbs[2]) << jnp.int32(14)

        t_t, cgt, sgt = descent(listt_v, m_t, pre_t, krem_t)
        S_top = sel_s_t + sgt + (krem_t - cgt).astype(jnp.float32) * _val_of_u(t_t)

        t_b, _, _ = descent(listb_v, m_b, pre_b, m_b - krem_b + 1)
        clt, slt = ltstats(listb_v, m_b, t_b)
        S_bot = sel_s_b + slt + (krem_b - clt).astype(jnp.float32) * _val_of_u(t_b)

        res = (S_top / jnp.float32(kmax)
               + S_bot * jnp.float32(_ALPHA) / jnp.float32(kmin)) * jnp.float32(0.5)
        outb_v[...] = res
        pltpu.sync_copy(outb_v, out_hbm.at[b, pl.ds(c0, _L)])
        return 0

    lax.fori_loop(0, gpw, group_body, 0)


def kernel(x):
    B, H, W, C = x.shape
    n = H * W
    kmax = _positive_k(_KFRAC, n)
    kmin = _positive_k(_KFRAC, n)
    xr = jnp.reshape(x, (B, n, C))

    mesh = plsc.VectorSubcoreMesh(core_axis_name="c", subcore_axis_name="s")
    body = functools.partial(_sc_pool_kernel_body, n, kmax, kmin, B, C)
    f = pl.kernel(
        body,
        mesh=mesh,
        out_type=jax.ShapeDtypeStruct((B, C), jnp.float32),
        scratch_types=[
            pltpu.VMEM((n, _L), jnp.float32),
            pltpu.VMEM((n, _L), jnp.int32),
            pltpu.VMEM((n, _L), jnp.int32),
            pltpu.VMEM((_NB, _L), jnp.int32),
            pltpu.VMEM((_NB, _L), jnp.float32),
            pltpu.VMEM((_L,), jnp.float32),
            pltpu.SemaphoreType.DMA,
        ],
        compiler_params=pltpu.CompilerParams(use_tc_tiling_on_sc=False,
                                             needs_layout_passes=False),
    )
    return f(xr)


# speedup vs baseline: 1.9020x; 1.9020x over previous
import functools

import jax
import jax.numpy as jnp
from jax import lax
from jax.experimental import pallas as pl
from jax.experimental.pallas import tpu as pltpu
from jax.experimental.pallas import tpu_sc as plsc

_ALPHA = 0.7
_KFRAC = 0.2
_L = 16
_NB = 64


def _positive_k(k, n):
    if k <= 0:
        return 0
    elif k < 1:
        return int(round(float(n) * float(k)))
    elif k > n:
        return int(n)
    else:
        return int(k)


def _keyu(v):
    i = lax.bitcast_convert_type(v, jnp.int32)
    return jnp.where(i < 0, ~i, i ^ jnp.int32(-2**31))


def _val_of_u(u):
    i = jnp.where(u < 0, u ^ jnp.int32(-2**31), ~u)
    return lax.bitcast_convert_type(i, jnp.float32)


def _digit(u, shift):
    s = jnp.full((_L,), shift, jnp.int32)
    return lax.shift_right_logical(u, s) & jnp.int32(_NB - 1)


def _sc_pool_kernel_body(n, kmax, kmin, B, C,
                         x_hbm, out_hbm,
                         data_v, listt_v, listb_v, hcnt_v, hsum_v, outb_v,
                         sem):
    lanes = lax.iota(jnp.int32, _L)
    zi = jnp.zeros((_L,), jnp.int32)
    zf = jnp.zeros((_L,), jnp.float32)
    ones_i = jnp.ones((_L,), jnp.int32)

    ncores = 2
    nsub = 16
    wid = lax.axis_index("c") * nsub + lax.axis_index("s")
    ngroups = B * (C // _L)
    gpw = ngroups // (ncores * nsub)

    def zero_h(d, _):
        hcnt_v[d] = zi
        hsum_v[d] = zf
        return 0
    lax.fori_loop(0, _NB, zero_h, 0)

    def scan_top(m_vec, krem):
        thresh = m_vec - krem + 1
        init = (zi, zf, zi > 0, zi, zi, zf, zi)

        @plsc.parallel_loop(0, _NB, unroll=4, carry=init)
        def body(d, carry):
            cum_c, cum_s, crossed, D, cntle, sumle, bcnt = carry
            c = hcnt_v[d]
            s = hsum_v[d]
            hcnt_v[d] = zi
            hsum_v[d] = zf
            cum_c = cum_c + c
            cum_s = cum_s + s
            newly = jnp.logical_and(jnp.logical_not(crossed), cum_c >= thresh)
            dv = zi + d
            D = jnp.where(newly, dv, D)
            cntle = jnp.where(newly, cum_c, cntle)
            sumle = jnp.where(newly, cum_s, sumle)
            bcnt = jnp.where(newly, c, bcnt)
            crossed = jnp.logical_or(crossed, newly)
            return cum_c, cum_s, crossed, D, cntle, sumle, bcnt

        cum_c, cum_s, _, D, cntle, sumle, bcnt = body
        add_cnt = m_vec - cntle
        add_sum = cum_s - sumle
        return D, add_cnt, add_sum, bcnt

    def scan_bot(m_vec, krem):
        thresh = krem
        init = (zi, zf, zi > 0, zi, zi, zf, zi)

        @plsc.parallel_loop(0, _NB, unroll=4, carry=init)
        def body(d, carry):
            cum_c, cum_s, crossed, D, cntlt, sumlt, bcnt = carry
            c = hcnt_v[d]
            s = hsum_v[d]
            hcnt_v[d] = zi
            hsum_v[d] = zf
            cum_c = cum_c + c
            cum_s = cum_s + s
            newly = jnp.logical_and(jnp.logical_not(crossed), cum_c >= thresh)
            dv = zi + d
            D = jnp.where(newly, dv, D)
            cntlt = jnp.where(newly, cum_c - c, cntlt)
            sumlt = jnp.where(newly, cum_s - s, sumlt)
            bcnt = jnp.where(newly, c, bcnt)
            crossed = jnp.logical_or(crossed, newly)
            return cum_c, cum_s, crossed, D, cntlt, sumlt, bcnt

        _, _, _, D, cntlt, sumlt, bcnt = body
        return D, cntlt, sumlt, bcnt

    def gather_row(list_v, j, m_vec):
        jv = zi + j
        valid = jv < m_vec
        idx = jnp.where(valid, list_v[j], 0)
        v = plsc.load_gather(data_v, [idx, lanes])
        return idx, v, valid

    def hist_list(list_v, m_vec, jmax, shift):
        @plsc.parallel_loop(0, jmax, unroll=4)
        def body(j):
            _, v, valid = gather_row(list_v, j, m_vec)
            u = _keyu(v)
            d = jnp.where(valid, _digit(u, shift), 0)
            plsc.addupdate_scatter(hcnt_v, [d, lanes],
                                   jnp.where(valid, 1, 0).astype(jnp.int32))
            plsc.addupdate_scatter(hsum_v, [d, lanes],
                                   jnp.where(valid, v, 0.0))

    def compact_list(list_v, m_vec, jmax, D, shift):
        def body(j, cur):
            idx, v, valid = gather_row(list_v, j, m_vec)
            u = _keyu(v)
            d = _digit(u, shift)
            mask = jnp.logical_and(valid, d == D)
            plsc.store_scatter(list_v, [cur, lanes], idx, mask=mask)
            return cur + jnp.where(mask, 1, 0)

        lax.fori_loop(0, jmax, body, zi)

    def descent(list_v, m_vec, jmax, prefix, krank):
        def bitbody(bi, t):
            bit = lax.shift_right_arithmetic(jnp.int32(1 << 13), bi)
            cand = t | bit

            @plsc.parallel_loop(0, jmax, unroll=4, carry=zi)
            def cnt(j, c):
                _, v, valid = gather_row(list_v, j, m_vec)
                u = _keyu(v)
                hit = jnp.logical_and(valid, u >= cand)
                return c + jnp.where(hit, 1, 0)

            return jnp.where(cnt >= krank, cand, t)

        t = lax.fori_loop(0, 14, bitbody, prefix)

        @plsc.parallel_loop(0, jmax, unroll=4, carry=(zi, zf))
        def gtstats(j, carry):
            cg, sg = carry
            _, v, valid = gather_row(list_v, j, m_vec)
            u = _keyu(v)
            g = jnp.logical_and(valid, u > t)
            return cg + jnp.where(g, 1, 0), sg + jnp.where(g, v, 0.0)

        cnt_gt, sum_gt = gtstats
        return t, cnt_gt, sum_gt

    def ltstats(list_v, m_vec, jmax, t):
        @plsc.parallel_loop(0, jmax, unroll=4, carry=(zi, zf))
        def body(j, carry):
            cl, sl = carry
            _, v, valid = gather_row(list_v, j, m_vec)
            u = _keyu(v)
            g = jnp.logical_and(valid, u < t)
            return cl + jnp.where(g, 1, 0), sl + jnp.where(g, v, 0.0)

        return body

    cpg = C // _L

    def group_body(g, _):
        gid = wid * gpw + g
        b = gid // cpg
        c0 = (gid % cpg) * _L

        pltpu.sync_copy(x_hbm.at[b, :, pl.ds(c0, _L)], data_v)

        @plsc.parallel_loop(0, n, unroll=8)
        def histA(i):
            v = data_v[i]
            u = _keyu(v)
            d = _digit(u, 26)
            plsc.addupdate_scatter(hcnt_v, [d, lanes], ones_i)
            plsc.addupdate_scatter(hsum_v, [d, lanes], v)

        m0 = zi + n
        k_t = zi + kmax
        k_b = zi + kmin
        D_t, add_c, add_s, bcnt_t = scan_top(m0, k_t)
        sel_c_t = add_c
        sel_s_t = add_s
        krem_t = k_t - add_c
        m_t = bcnt_t
        D_b, cntlt, sumlt, bcnt_b = scan_bot(m0, k_b)
        sel_c_b = cntlt
        sel_s_b = sumlt
        krem_b = k_b - cntlt
        m_b = bcnt_b

        @plsc.parallel_loop(0, n, unroll=8, carry=(zi, zi))
        def compactC(i, carry):
            cur_t, cur_b = carry
            v = data_v[i]
            u = _keyu(v)
            d = _digit(u, 26)
            iv = zi + i
            mt = d == D_t
            plsc.store_scatter(listt_v, [cur_t, lanes], iv, mask=mt)
            mb = d == D_b
            plsc.store_scatter(listb_v, [cur_b, lanes], iv, mask=mb)
            return cur_t + jnp.where(mt, 1, 0), cur_b + jnp.where(mb, 1, 0)

        Dt1, Db1 = D_t, D_b
        Dts = [Dt1]
        Dbs = [Db1]
        for shift in (20, 14):
            jmax_t = jnp.max(m_t)
            hist_list(listt_v, m_t, jmax_t, shift)
            D, add_c, add_s, bcnt = scan_top(m_t, krem_t)
            sel_c_t = sel_c_t + add_c
            sel_s_t = sel_s_t + add_s
            krem_t = krem_t - add_c
            compact_list(listt_v, m_t, jmax_t, D, shift)
            m_t = bcnt
            Dts.append(D)

            jmax_b = jnp.max(m_b)
            hist_list(listb_v, m_b, jmax_b, shift)
            D, cntlt, sumlt, bcnt = scan_bot(m_b, krem_b)
            sel_c_b = sel_c_b + cntlt
            sel_s_b = sel_s_b + sumlt
            krem_b = krem_b - cntlt
            compact_list(listb_v, m_b, jmax_b, D, shift)
            m_b = bcnt
            Dbs.append(D)

        six = jnp.int32(6)
        pre_t = ((Dts[0] << six | Dts[1]) << six | Dts[2]) << jnp.int32(14)
        pre_b = ((Dbs[0] << six | Dbs[1]) << six | Dbs[2]) << jnp.int32(14)

        jmax_t = jnp.max(m_t)
        jmax_b = jnp.max(m_b)
        t_t, cgt, sgt = descent(listt_v, m_t, jmax_t, pre_t, krem_t)
        S_top = sel_s_t + sgt + (krem_t - cgt).astype(jnp.float32) * _val_of_u(t_t)

        t_b, _, _ = descent(listb_v, m_b, jmax_b, pre_b, m_b - krem_b + 1)
        clt, slt = ltstats(listb_v, m_b, jmax_b, t_b)
        S_bot = sel_s_b + slt + (krem_b - clt).astype(jnp.float32) * _val_of_u(t_b)

        res = (S_top / jnp.float32(kmax)
               + S_bot * jnp.float32(_ALPHA) / jnp.float32(kmin)) * jnp.float32(0.5)
        outb_v[...] = res
        pltpu.sync_copy(outb_v, out_hbm.at[b, pl.ds(c0, _L)])
        return 0

    lax.fori_loop(0, gpw, group_body, 0)


def kernel(x):
    B, H, W, C = x.shape
    n = H * W
    kmax = _positive_k(_KFRAC, n)
    kmin = _positive_k(_KFRAC, n)
    xr = jnp.reshape(x, (B, n, C))

    mesh = plsc.VectorSubcoreMesh(core_axis_name="c", subcore_axis_name="s")
    body = functools.partial(_sc_pool_kernel_body, n, kmax, kmin, B, C)
    f = pl.kernel(
        body,
        mesh=mesh,
        out_type=jax.ShapeDtypeStruct((B, C), jnp.float32),
        scratch_types=[
            pltpu.VMEM((n, _L), jnp.float32),
            pltpu.VMEM((n, _L), jnp.int32),
            pltpu.VMEM((n, _L), jnp.int32),
            pltpu.VMEM((_NB, _L), jnp.int32),
            pltpu.VMEM((_NB, _L), jnp.float32),
            pltpu.VMEM((_L,), jnp.float32),
            pltpu.SemaphoreType.DMA,
        ],
        compiler_params=pltpu.CompilerParams(use_tc_tiling_on_sc=False,
                                             needs_layout_passes=False),
    )
    return f(xr)


# speedup vs baseline: 3.0897x; 1.6244x over previous
import functools

import jax
import jax.numpy as jnp
from jax import lax
from jax.experimental import pallas as pl
from jax.experimental.pallas import tpu as pltpu
from jax.experimental.pallas import tpu_sc as plsc

_ALPHA = 0.7
_KFRAC = 0.2
_L = 16
_NB = 64


def _positive_k(k, n):
    if k <= 0:
        return 0
    elif k < 1:
        return int(round(float(n) * float(k)))
    elif k > n:
        return int(n)
    else:
        return int(k)


def _keyu(v):
    i = lax.bitcast_convert_type(v, jnp.int32)
    return jnp.where(i < 0, ~i, i ^ jnp.int32(-2**31))


def _val_of_u(u):
    i = jnp.where(u < 0, u ^ jnp.int32(-2**31), ~u)
    return lax.bitcast_convert_type(i, jnp.float32)


def _digit(u, shift):
    s = jnp.full((_L,), shift, jnp.int32)
    return lax.shift_right_logical(u, s) & jnp.int32(_NB - 1)


def _sc_pool_kernel_body(n, kmax, kmin, B, C,
                         x_hbm, out_hbm,
                         data_v, listt_v, listb_v, listt2_v, listb2_v,
                         hcnt_v, hsum_v, outb_v, sem):
    lanes = lax.iota(jnp.int32, _L)
    zi = jnp.zeros((_L,), jnp.int32)
    zf = jnp.zeros((_L,), jnp.float32)
    ones_i = jnp.ones((_L,), jnp.int32)

    ncores = 2
    nsub = 16
    wid = lax.axis_index("c") * nsub + lax.axis_index("s")
    ngroups = B * (C // _L)
    gpw = ngroups // (ncores * nsub)

    def zero_h(d, _):
        hcnt_v[d] = zi
        hsum_v[d] = zf
        return 0
    lax.fori_loop(0, _NB, zero_h, 0)

    def scan_top(m_vec, krem):
        thresh = m_vec - krem + 1
        init = (zi, zf, zi > 0, zi, zi, zf, zi)

        @plsc.parallel_loop(0, _NB, unroll=4, carry=init)
        def body(d, carry):
            cum_c, cum_s, crossed, D, cntle, sumle, bcnt = carry
            c = hcnt_v[d]
            s = hsum_v[d]
            hcnt_v[d] = zi
            hsum_v[d] = zf
            cum_c = cum_c + c
            cum_s = cum_s + s
            newly = jnp.logical_and(jnp.logical_not(crossed), cum_c >= thresh)
            dv = zi + d
            D = jnp.where(newly, dv, D)
            cntle = jnp.where(newly, cum_c, cntle)
            sumle = jnp.where(newly, cum_s, sumle)
            bcnt = jnp.where(newly, c, bcnt)
            crossed = jnp.logical_or(crossed, newly)
            return cum_c, cum_s, crossed, D, cntle, sumle, bcnt

        cum_c, cum_s, _, D, cntle, sumle, bcnt = body
        add_cnt = m_vec - cntle
        add_sum = cum_s - sumle
        return D, add_cnt, add_sum, bcnt

    def scan_bot(m_vec, krem):
        thresh = krem
        init = (zi, zf, zi > 0, zi, zi, zf, zi)

        @plsc.parallel_loop(0, _NB, unroll=4, carry=init)
        def body(d, carry):
            cum_c, cum_s, crossed, D, cntlt, sumlt, bcnt = carry
            c = hcnt_v[d]
            s = hsum_v[d]
            hcnt_v[d] = zi
            hsum_v[d] = zf
            cum_c = cum_c + c
            cum_s = cum_s + s
            newly = jnp.logical_and(jnp.logical_not(crossed), cum_c >= thresh)
            dv = zi + d
            D = jnp.where(newly, dv, D)
            cntlt = jnp.where(newly, cum_c - c, cntlt)
            sumlt = jnp.where(newly, cum_s - s, sumlt)
            bcnt = jnp.where(newly, c, bcnt)
            crossed = jnp.logical_or(crossed, newly)
            return cum_c, cum_s, crossed, D, cntlt, sumlt, bcnt

        _, _, _, D, cntlt, sumlt, bcnt = body
        return D, cntlt, sumlt, bcnt

    def gather_row(buf, list_v, j, m_vec):
        jv = zi + j
        valid = jv < m_vec
        idx = jnp.where(valid, list_v[j], 0)
        v = plsc.load_gather(data_v, [buf, idx, lanes])
        return idx, v, valid

    def hist_list(buf, list_v, m_vec, jmax, shift):
        @plsc.parallel_loop(0, jmax, unroll=4)
        def body(j):
            _, v, valid = gather_row(buf, list_v, j, m_vec)
            u = _keyu(v)
            d = jnp.where(valid, _digit(u, shift), 0)
            plsc.addupdate_scatter(hcnt_v, [d, lanes],
                                   jnp.where(valid, 1, 0).astype(jnp.int32))
            plsc.addupdate_scatter(hsum_v, [d, lanes],
                                   jnp.where(valid, v, 0.0))

    def compact_list(buf, src_v, dst_v, m_vec, jmax, D, shift):
        @plsc.parallel_loop(0, jmax, unroll=4, carry=zi)
        def body(j, cur):
            idx, v, valid = gather_row(buf, src_v, j, m_vec)
            u = _keyu(v)
            d = _digit(u, shift)
            mask = jnp.logical_and(valid, d == D)
            plsc.store_scatter(dst_v, [cur, lanes], idx, mask=mask)
            return cur + jnp.where(mask, 1, 0)

    def descent(buf, list_v, m_vec, jmax, prefix, krank):
        def bitbody(bi, t):
            bit = lax.shift_right_arithmetic(jnp.int32(1 << 13), bi)
            cand = t | bit

            @plsc.parallel_loop(0, jmax, unroll=4, carry=zi)
            def cnt(j, c):
                _, v, valid = gather_row(buf, list_v, j, m_vec)
                u = _keyu(v)
                hit = jnp.logical_and(valid, u >= cand)
                return c + jnp.where(hit, 1, 0)

            return jnp.where(cnt >= krank, cand, t)

        t = lax.fori_loop(0, 14, bitbody, prefix)

        @plsc.parallel_loop(0, jmax, unroll=4, carry=(zi, zf))
        def gtstats(j, carry):
            cg, sg = carry
            _, v, valid = gather_row(buf, list_v, j, m_vec)
            u = _keyu(v)
            g = jnp.logical_and(valid, u > t)
            return cg + jnp.where(g, 1, 0), sg + jnp.where(g, v, 0.0)

        cnt_gt, sum_gt = gtstats
        return t, cnt_gt, sum_gt

    def ltstats(buf, list_v, m_vec, jmax, t):
        @plsc.parallel_loop(0, jmax, unroll=4, carry=(zi, zf))
        def body(j, carry):
            cl, sl = carry
            _, v, valid = gather_row(buf, list_v, j, m_vec)
            u = _keyu(v)
            g = jnp.logical_and(valid, u < t)
            return cl + jnp.where(g, 1, 0), sl + jnp.where(g, v, 0.0)

        return body

    cpg = C // _L

    def dma_start(g):
        gid = wid * gpw + jnp.minimum(g, gpw - 1)
        b = gid // cpg
        c0 = (gid % cpg) * _L
        pltpu.async_copy(x_hbm.at[b, :, pl.ds(c0, _L)],
                         data_v.at[lax.rem(g, 2)], sem)

    dma_start(0)

    def group_body(g, _):
        gid = wid * gpw + g
        b = gid // cpg
        c0 = (gid % cpg) * _L
        buf = lax.rem(g, 2)
        bufv = zi + buf

        pltpu.make_async_copy(x_hbm.at[0, :, pl.ds(0, _L)],
                              data_v.at[buf], sem).wait()
        dma_start(g + 1)

        @plsc.parallel_loop(0, n, unroll=8)
        def histA(i):
            v = data_v[buf, i]
            u = _keyu(v)
            d = _digit(u, 26)
            plsc.addupdate_scatter(hcnt_v, [d, lanes], ones_i)
            plsc.addupdate_scatter(hsum_v, [d, lanes], v)

        m0 = zi + n
        k_t = zi + kmax
        k_b = zi + kmin
        D_t, add_c, add_s, bcnt_t = scan_top(m0, k_t)
        sel_c_t = add_c
        sel_s_t = add_s
        krem_t = k_t - add_c
        m_t = bcnt_t
        D_b, cntlt, sumlt, bcnt_b = scan_bot(m0, k_b)
        sel_c_b = cntlt
        sel_s_b = sumlt
        krem_b = k_b - cntlt
        m_b = bcnt_b

        @plsc.parallel_loop(0, n, unroll=8, carry=(zi, zi))
        def compactC(i, carry):
            cur_t, cur_b = carry
            v = data_v[buf, i]
            u = _keyu(v)
            d = _digit(u, 26)
            iv = zi + i
            mt = d == D_t
            plsc.store_scatter(listt_v, [cur_t, lanes], iv, mask=mt)
            mb = d == D_b
            plsc.store_scatter(listb_v, [cur_b, lanes], iv, mask=mb)
            return cur_t + jnp.where(mt, 1, 0), cur_b + jnp.where(mb, 1, 0)

        Dt1, Db1 = D_t, D_b
        Dts = [Dt1]
        Dbs = [Db1]
        srcs = ((listt_v, listb_v), (listt2_v, listb2_v))
        for ri, shift in enumerate((20, 14)):
            src_t, src_b = srcs[ri % 2]
            dst_t, dst_b = srcs[(ri + 1) % 2]
            jmax_t = jnp.max(m_t)
            hist_list(bufv, src_t, m_t, jmax_t, shift)
            D, add_c, add_s, bcnt = scan_top(m_t, krem_t)
            sel_c_t = sel_c_t + add_c
            sel_s_t = sel_s_t + add_s
            krem_t = krem_t - add_c
            compact_list(bufv, src_t, dst_t, m_t, jmax_t, D, shift)
            m_t = bcnt
            Dts.append(D)

            jmax_b = jnp.max(m_b)
            hist_list(bufv, src_b, m_b, jmax_b, shift)
            D, cntlt, sumlt, bcnt = scan_bot(m_b, krem_b)
            sel_c_b = sel_c_b + cntlt
            sel_s_b = sel_s_b + sumlt
            krem_b = krem_b - cntlt
            compact_list(bufv, src_b, dst_b, m_b, jmax_b, D, shift)
            m_b = bcnt
            Dbs.append(D)

        six = jnp.int32(6)
        pre_t = ((Dts[0] << six | Dts[1]) << six | Dts[2]) << jnp.int32(14)
        pre_b = ((Dbs[0] << six | Dbs[1]) << six | Dbs[2]) << jnp.int32(14)

        jmax_t = jnp.max(m_t)
        jmax_b = jnp.max(m_b)
        t_t, cgt, sgt = descent(bufv, listt_v, m_t, jmax_t, pre_t, krem_t)
        S_top = sel_s_t + sgt + (krem_t - cgt).astype(jnp.float32) * _val_of_u(t_t)

        t_b, _, _ = descent(bufv, listb_v, m_b, jmax_b, pre_b, m_b - krem_b + 1)
        clt, slt = ltstats(bufv, listb_v, m_b, jmax_b, t_b)
        S_bot = sel_s_b + slt + (krem_b - clt).astype(jnp.float32) * _val_of_u(t_b)

        res = (S_top / jnp.float32(kmax)
               + S_bot * jnp.float32(_ALPHA) / jnp.float32(kmin)) * jnp.float32(0.5)
        outb_v[...] = res
        pltpu.sync_copy(outb_v, out_hbm.at[b, pl.ds(c0, _L)])
        return 0

    lax.fori_loop(0, gpw, group_body, 0)


def kernel(x):
    B, H, W, C = x.shape
    n = H * W
    kmax = _positive_k(_KFRAC, n)
    kmin = _positive_k(_KFRAC, n)
    xr = jnp.reshape(x, (B, n, C))

    mesh = plsc.VectorSubcoreMesh(core_axis_name="c", subcore_axis_name="s")
    body = functools.partial(_sc_pool_kernel_body, n, kmax, kmin, B, C)
    f = pl.kernel(
        body,
        mesh=mesh,
        out_type=jax.ShapeDtypeStruct((B, C), jnp.float32),
        scratch_types=[
            pltpu.VMEM((2, n, _L), jnp.float32),
            pltpu.VMEM((n, _L), jnp.int32),
            pltpu.VMEM((n, _L), jnp.int32),
            pltpu.VMEM((n, _L), jnp.int32),
            pltpu.VMEM((n, _L), jnp.int32),
            pltpu.VMEM((_NB, _L), jnp.int32),
            pltpu.VMEM((_NB, _L), jnp.float32),
            pltpu.VMEM((_L,), jnp.float32),
            pltpu.SemaphoreType.DMA,
        ],
        compiler_params=pltpu.CompilerParams(use_tc_tiling_on_sc=False,
                                             needs_layout_passes=False),
    )
    return f(xr)
